# baseline (speedup 1.0000x reference)
import functools

import jax
import jax.numpy as jnp
from jax import lax
from jax.experimental import pallas as pl
from jax.experimental.pallas import tpu as pltpu
from jax.experimental.pallas import tpu_sc as plsc

_EMBED_DIM = 64
_CHUNK = 128

_info = plsc.get_sparse_core_info()
_NC, _NS = _info.num_cores, _info.num_subcores
_NW = _NC * _NS


@functools.partial(jax.jit, static_argnames=("rows_per_w",))
def _sc_gather(table, idx2d, rows_per_w):
    n_idx_rows = idx2d.shape[0]
    d = table.shape[1]
    mesh = plsc.VectorSubcoreMesh(core_axis_name="c", subcore_axis_name="s")

    @functools.partial(
        pl.kernel,
        mesh=mesh,
        compiler_params=pltpu.CompilerParams(use_tc_tiling_on_sc=False),
        out_type=jax.ShapeDtypeStruct((n_idx_rows * _CHUNK, d), jnp.float32),
        scratch_types=[
            pltpu.VMEM((rows_per_w, _CHUNK), jnp.int32),
            pltpu.VMEM((_CHUNK, d), jnp.float32),
            pltpu.SemaphoreType.DMA,
        ],
    )
    def k(table_hbm, idx_hbm, out_hbm, idx_v, rows_v, sem):
        wid = lax.axis_index("s") * _NC + lax.axis_index("c")
        base = wid * rows_per_w
        pltpu.sync_copy(idx_hbm.at[pl.ds(base, rows_per_w)], idx_v)

        def step(j, carry):
            pltpu.async_copy(table_hbm.at[idx_v.at[j]], rows_v, sem).wait()
            pltpu.sync_copy(rows_v, out_hbm.at[pl.ds((base + j) * _CHUNK, _CHUNK)])
            return carry

        lax.fori_loop(0, rows_per_w, step, 0)

    return k(table, idx2d)


def kernel(token_ids, embedding_matrix):
    b, t = token_ids.shape
    d = embedding_matrix.shape[1]
    n = b * t
    idx2d = token_ids.astype(jnp.int32).reshape(n // _CHUNK, _CHUNK)
    rows_per_w = idx2d.shape[0] // _NW
    out = _sc_gather(embedding_matrix, idx2d, rows_per_w)
    return out.reshape(b, t, d)


# speedup vs baseline: 1.1166x; 1.1166x over previous
import functools

import jax
import jax.numpy as jnp
from jax import lax
from jax.experimental import pallas as pl
from jax.experimental.pallas import tpu as pltpu
from jax.experimental.pallas import tpu_sc as plsc

_EMBED_DIM = 64
_CHUNK = 128

_info = plsc.get_sparse_core_info()
_NC, _NS = _info.num_cores, _info.num_subcores
_NW = _NC * _NS


@functools.partial(jax.jit, static_argnames=("rows_per_w",))
def _sc_gather(table, idx2d, rows_per_w):
    n_idx_rows = idx2d.shape[0]
    d = table.shape[1]
    mesh = plsc.VectorSubcoreMesh(core_axis_name="c", subcore_axis_name="s")

    nbuf = 4
    ngroups = rows_per_w // nbuf

    @functools.partial(
        pl.kernel,
        mesh=mesh,
        compiler_params=pltpu.CompilerParams(use_tc_tiling_on_sc=False),
        out_type=jax.ShapeDtypeStruct((n_idx_rows * _CHUNK, d), jnp.float32),
        scratch_types=[
            pltpu.VMEM((rows_per_w, _CHUNK), jnp.int32),
            pltpu.VMEM((2, nbuf, _CHUNK, d), jnp.float32),
            pltpu.SemaphoreType.DMA((2, nbuf)),
            pltpu.SemaphoreType.DMA((2, nbuf)),
        ],
    )
    def k(table_hbm, idx_hbm, out_hbm, idx_v, rows_v, gsem, ssem):
        wid = lax.axis_index("s") * _NC + lax.axis_index("c")
        base = wid * rows_per_w
        pltpu.sync_copy(idx_hbm.at[pl.ds(base, rows_per_w)], idx_v)

        def gather(g, p, b):
            j = g * nbuf + b
            return pltpu.make_async_copy(
                table_hbm.at[idx_v.at[j]], rows_v.at[p, b], gsem.at[p, b])

        def store(g, p, b):
            j = g * nbuf + b
            return pltpu.make_async_copy(
                rows_v.at[p, b],
                out_hbm.at[pl.ds((base + j) * _CHUNK, _CHUNK)],
                ssem.at[p, b])

        for p in (0, 1):
            for b in range(nbuf):
                gather(p, p, b).start()

        def outer(gg, carry):
            for p in (0, 1):
                g = gg * 2 + p
                for b in range(nbuf):
                    gather(g, p, b).wait()
                    store(g, p, b).start()
                ng = g + 2

                @pl.when(ng < ngroups)
                def _():
                    for b in range(nbuf):
                        store(g, p, b).wait()
                        gather(ng, p, b).start()

            return carry

        lax.fori_loop(0, ngroups // 2, outer, 0)

        for p in (0, 1):
            g = ngroups - 2 + p
            for b in range(nbuf):
                store(g, p, b).wait()

    return k(table, idx2d)


def kernel(token_ids, embedding_matrix):
    b, t = token_ids.shape
    d = embedding_matrix.shape[1]
    n = b * t
    idx2d = token_ids.astype(jnp.int32).reshape(n // _CHUNK, _CHUNK)
    rows_per_w = idx2d.shape[0] // _NW
    out = _sc_gather(embedding_matrix, idx2d, rows_per_w)
    return out.reshape(b, t, d)


# speedup vs baseline: 1.1176x; 1.0009x over previous
import functools

import jax
import jax.numpy as jnp
from jax import lax
from jax.experimental import pallas as pl
from jax.experimental.pallas import tpu as pltpu
from jax.experimental.pallas import tpu_sc as plsc

_info = plsc.get_sparse_core_info()
_NC, _NS = _info.num_cores, _info.num_subcores
_NW = _NC * _NS

_SPLIT = (104, 96)


def _sc_gather(table, token_ids):
    b, t = token_ids.shape
    d = table.shape[1]
    rows_per_w = b // _NW
    nbuf = 2
    ngroups = rows_per_w // nbuf
    mesh = plsc.VectorSubcoreMesh(core_axis_name="c", subcore_axis_name="s")

    @functools.partial(
        pl.kernel,
        mesh=mesh,
        compiler_params=pltpu.CompilerParams(use_tc_tiling_on_sc=False),
        out_type=jax.ShapeDtypeStruct((b, t, d), jnp.float32),
        scratch_types=[
            pltpu.VMEM((rows_per_w, t), jnp.int32),
            pltpu.VMEM((2, nbuf, 2, _SPLIT[0], d), jnp.float32),
            pltpu.SemaphoreType.DMA((2, nbuf, 2)),
            pltpu.SemaphoreType.DMA((2, nbuf, 2)),
        ],
    )
    def k(table_hbm, idx_hbm, out_hbm, idx_v, rows_v, gsem, ssem):
        wid = lax.axis_index("s") * _NC + lax.axis_index("c")
        base = wid * rows_per_w
        pltpu.sync_copy(idx_hbm.at[pl.ds(base, rows_per_w)], idx_v)

        def gather(g, p, j, h):
            ln = _SPLIT[h]
            off = h * _SPLIT[0]
            row = g * nbuf + j
            return pltpu.make_async_copy(
                table_hbm.at[idx_v.at[row, pl.ds(off, ln)]],
                rows_v.at[p, j, h, pl.ds(0, ln)],
                gsem.at[p, j, h])

        def store(g, p, j, h):
            ln = _SPLIT[h]
            off = h * _SPLIT[0]
            row = g * nbuf + j
            return pltpu.make_async_copy(
                rows_v.at[p, j, h, pl.ds(0, ln)],
                out_hbm.at[base + row, pl.ds(off, ln)],
                ssem.at[p, j, h])

        for p in (0, 1):
            for j in range(nbuf):
                for h in (0, 1):
                    gather(p, p, j, h).start()

        def outer(gg, carry):
            for p in (0, 1):
                g = gg * 2 + p
                for j in range(nbuf):
                    for h in (0, 1):
                        gather(g, p, j, h).wait()
                        store(g, p, j, h).start()
                ng = g + 2

                @pl.when(ng < ngroups)
                def _():
                    for j in range(nbuf):
                        for h in (0, 1):
                            store(g, p, j, h).wait()
                            gather(ng, p, j, h).start()

            return carry

        lax.fori_loop(0, ngroups // 2, outer, 0)

        for p in (0, 1):
            g = ngroups - 2 + p
            for j in range(nbuf):
                for h in (0, 1):
                    store(g, p, j, h).wait()

    return k(table, token_ids)


def kernel(token_ids, embedding_matrix):
    return _sc_gather(embedding_matrix, token_ids.astype(jnp.int32))


# speedup vs baseline: 1.3656x; 1.2219x over previous
import functools

import jax
import jax.numpy as jnp
from jax import lax
from jax.experimental import pallas as pl
from jax.experimental.pallas import tpu as pltpu
from jax.experimental.pallas import tpu_sc as plsc

_info = plsc.get_sparse_core_info()
_NC, _NS = _info.num_cores, _info.num_subcores
_NW = _NC * _NS

_DP = 128

_SPLIT = (104, 96)
_NRING = 3


def _sc_gather(table_padded, idx_flat, b, t):
    rows_per_w = b // _NW
    toks_per_w = rows_per_w * t
    mesh = plsc.VectorSubcoreMesh(core_axis_name="c", subcore_axis_name="s")

    @functools.partial(
        pl.kernel,
        mesh=mesh,
        out_type=jax.ShapeDtypeStruct((b, t, _DP), jnp.float32),
        scratch_types=[
            pltpu.VMEM((toks_per_w,), jnp.int32),
            pltpu.VMEM((_NRING, 2, _SPLIT[0], _DP), jnp.float32),
            pltpu.SemaphoreType.DMA((_NRING, 2)),
            pltpu.SemaphoreType.DMA((_NRING, 2)),
        ],
    )
    def k(table_hbm, idx_hbm, out_hbm, idx_v, rows_v, gsem, ssem):
        wid = lax.axis_index("s") * _NC + lax.axis_index("c")
        base = wid * rows_per_w
        pltpu.sync_copy(idx_hbm.at[pl.ds(base * t, toks_per_w)], idx_v)

        def gather(g, p, h):
            ln = _SPLIT[h]
            off = g * t + h * _SPLIT[0]
            return pltpu.make_async_copy(
                table_hbm.at[idx_v.at[pl.ds(off, ln)]],
                rows_v.at[p, h, pl.ds(0, ln)],
                gsem.at[p, h])

        def store(g, p, h):
            ln = _SPLIT[h]
            return pltpu.make_async_copy(
                rows_v.at[p, h, pl.ds(0, ln)],
                out_hbm.at[base + g, pl.ds(h * _SPLIT[0], ln)],
                ssem.at[p, h])

        nfull = (rows_per_w // _NRING) * _NRING

        for p in range(_NRING):
            for h in (0, 1):
                gather(p, p, h).start()

        def outer(gg, carry):
            for p in range(_NRING):
                g = gg * _NRING + p
                for h in (0, 1):
                    gather(g, p, h).wait()
                    store(g, p, h).start()
                ng = g + _NRING

                @pl.when(ng < rows_per_w)
                def _():
                    for h in (0, 1):
                        store(g, p, h).wait()
                        gather(ng, p, h).start()

            return carry

        lax.fori_loop(0, rows_per_w // _NRING, outer, 0)

        for g in range(nfull, rows_per_w):
            p = g % _NRING
            for h in (0, 1):
                gather(g, p, h).wait()
                store(g, p, h).start()

        for g in range(rows_per_w - _NRING, rows_per_w):
            p = g % _NRING
            for h in (0, 1):
                store(g, p, h).wait()

    return k(table_padded, idx_flat)


def kernel(token_ids, embedding_matrix):
    b, t = token_ids.shape
    d = embedding_matrix.shape[1]
    table_padded = jnp.pad(embedding_matrix, ((0, 0), (0, _DP - d)))
    idx_flat = token_ids.astype(jnp.int32).reshape(-1)
    out_padded = _sc_gather(table_padded, idx_flat, b, t)
    return out_padded[:, :, :d]
